# Initial kernel scaffold; baseline (speedup 1.0000x reference)
import jax
import jax.numpy as jnp
from jax.experimental import pallas as pl


def kernel(x, edge_index, edge_attr, W1, b1, g1, be1, W2, b2, g2, be2, Wf1, bf1, Wf2, bf2):
    raise NotImplementedError("write your pallas kernel here")


# speedup vs baseline: 1.6591x; 1.6591x over previous
import functools

import jax
import jax.numpy as jnp
from jax.experimental import pallas as pl
from jax.experimental.pallas import tpu as pltpu

_EPS = 1e-5


def _fused_mlp_kernel(x_ref, W1_ref, g1_ref, be1_ref, W2_ref, g2_ref, be2_ref,
                      Wf1_ref, bf1_ref, Wf2_ref, bf2_ref, out_ref,
                      h_scr, s_ref, q_ref, sc_ref, sh_ref,
                      *, n_rows, bm):
    p = pl.program_id(0)
    i = pl.program_id(1)
    rows = pl.ds(i * bm, bm)
    inv_n = 1.0 / n_rows

    @pl.when(p == 0)
    def _phase0():
        @pl.when(i == 0)
        def _init():
            s_ref[...] = jnp.zeros_like(s_ref)
            q_ref[...] = jnp.zeros_like(q_ref)

        u = jnp.dot(x_ref[...], W1_ref[...], preferred_element_type=jnp.float32)
        h_scr[rows, :] = u
        s_ref[...] += jnp.sum(u, axis=0, keepdims=True)
        q_ref[...] += jnp.sum(u * u, axis=0, keepdims=True)

    @pl.when(p == 1)
    def _phase1():
        @pl.when(i == 0)
        def _finalize_bn1():
            mean = s_ref[...] * inv_n
            var = q_ref[...] * inv_n - mean * mean
            scale = g1_ref[...] * jax.lax.rsqrt(var + _EPS)
            sc_ref[...] = scale
            sh_ref[...] = be1_ref[...] - mean * scale
            s_ref[...] = jnp.zeros_like(s_ref)
            q_ref[...] = jnp.zeros_like(q_ref)

        u = h_scr[rows, :]
        h1 = jnp.maximum(u * sc_ref[...] + sh_ref[...], 0.0)
        h2 = jnp.dot(h1, W2_ref[...], preferred_element_type=jnp.float32)
        h_scr[rows, :] = h2
        s_ref[...] += jnp.sum(h2, axis=0, keepdims=True)
        q_ref[...] += jnp.sum(h2 * h2, axis=0, keepdims=True)

    @pl.when(p == 2)
    def _phase2():
        @pl.when(i == 0)
        def _finalize_bn2():
            mean = s_ref[...] * inv_n
            var = q_ref[...] * inv_n - mean * mean
            scale = g2_ref[...] * jax.lax.rsqrt(var + _EPS)
            sc_ref[...] = scale
            sh_ref[...] = be2_ref[...] - mean * scale

        h2 = h_scr[rows, :]
        hn = h2 * sc_ref[...] + sh_ref[...]
        m = jnp.dot(hn, Wf1_ref[...], preferred_element_type=jnp.float32)
        m = jnp.maximum(m + bf1_ref[...], 0.0)
        out_ref[...] = jnp.dot(m, Wf2_ref[...],
                               preferred_element_type=jnp.float32) + bf2_ref[...]


def kernel(x, edge_index, edge_attr, W1, b1, g1, be1, W2, b2, g2, be2,
           Wf1, bf1, Wf2, bf2):
    del edge_index, edge_attr, b1, b2
    n, f_in = x.shape
    h_dim = W1.shape[1]
    mid = Wf1.shape[1]
    out_c = Wf2.shape[1]

    bm = 2000
    nb = n // bm

    full = lambda shape: pl.BlockSpec(shape, lambda p, i: (0, 0))
    row2 = lambda f: (1, f)

    grid = (3, nb)
    body = functools.partial(_fused_mlp_kernel, n_rows=n, bm=bm)
    out = pl.pallas_call(
        body,
        grid=grid,
        in_specs=[
            pl.BlockSpec((bm, f_in), lambda p, i: (i, 0)),
            full((f_in, h_dim)),
            full(row2(h_dim)),
            full(row2(h_dim)),
            full((h_dim, h_dim)),
            full(row2(h_dim)),
            full(row2(h_dim)),
            full((h_dim, mid)),
            full(row2(mid)),
            full((mid, out_c)),
            full(row2(out_c)),
        ],
        out_specs=pl.BlockSpec((bm, out_c), lambda p, i: (i, 0)),
        out_shape=jax.ShapeDtypeStruct((n, out_c), jnp.float32),
        scratch_shapes=[
            pltpu.VMEM((n, h_dim), jnp.float32),
            pltpu.VMEM((1, h_dim), jnp.float32),
            pltpu.VMEM((1, h_dim), jnp.float32),
            pltpu.VMEM((1, h_dim), jnp.float32),
            pltpu.VMEM((1, h_dim), jnp.float32),
        ],
        compiler_params=pltpu.CompilerParams(
            dimension_semantics=("arbitrary", "arbitrary"),
        ),
    )(
        x, W1, g1.reshape(1, -1), be1.reshape(1, -1),
        W2, g2.reshape(1, -1), be2.reshape(1, -1),
        Wf1, bf1.reshape(1, -1), Wf2, bf2.reshape(1, -1),
    )
    return out
